# Initial kernel scaffold; baseline (speedup 1.0000x reference)
import jax
import jax.numpy as jnp
from jax.experimental import pallas as pl


def kernel(prediction, target):
    raise NotImplementedError("write your pallas kernel here")


# speedup vs baseline: 1.1711x; 1.1711x over previous
import jax
import jax.numpy as jnp
from jax.experimental import pallas as pl
from jax.experimental.pallas import tpu as pltpu

_ROWS = 2 * 8192
_COLS = 2048
_BLOCK_ROWS = 1024


def _mse_kernel(p_ref, t_ref, out_ref, acc_ref):
    i = pl.program_id(0)
    n = pl.num_programs(0)
    p = p_ref[...]
    t = t_ref[...]
    d = p - t
    sq = d * d
    mask = t != 0.0
    s = jnp.sum(jnp.where(mask, sq, 0.0))
    c = jnp.sum(jnp.where(mask, 1.0, 0.0))

    @pl.when(i == 0)
    def _init():
        acc_ref[0] = 0.0
        acc_ref[1] = 0.0

    acc_ref[0] += s
    acc_ref[1] += c

    @pl.when(i == n - 1)
    def _fini():
        out_ref[0] = acc_ref[0] / acc_ref[1]


def kernel(prediction, target):
    p = prediction.reshape(_ROWS, _COLS)
    t = target.reshape(_ROWS, _COLS)
    grid = _ROWS // _BLOCK_ROWS
    out = pl.pallas_call(
        _mse_kernel,
        grid=(grid,),
        in_specs=[
            pl.BlockSpec((_BLOCK_ROWS, _COLS), lambda i: (i, 0)),
            pl.BlockSpec((_BLOCK_ROWS, _COLS), lambda i: (i, 0)),
        ],
        out_specs=pl.BlockSpec(memory_space=pltpu.SMEM),
        out_shape=jax.ShapeDtypeStruct((1,), jnp.float32),
        scratch_shapes=[pltpu.SMEM((2,), jnp.float32)],
    )(p, t)
    return out[0]
